# Initial kernel scaffold; baseline (speedup 1.0000x reference)
import jax
import jax.numpy as jnp
from jax.experimental import pallas as pl


def kernel(x, table):
    raise NotImplementedError("write your pallas kernel here")


# speedup vs baseline: 1.1013x; 1.1013x over previous
import functools

import jax
import jax.numpy as jnp
from jax import lax
from jax.experimental import pallas as pl
from jax.experimental.pallas import tpu as pltpu
from jax.experimental.pallas import tpu_sc as plsc

_BATCH = 16384
_HIST = 50
_D = 32
_B = _BATCH * _HIST

_INFO = plsc.get_sparse_core_info()
_NC = _INFO.num_cores
_NS = _INFO.num_subcores
_NW = _NC * _NS
_BPW = _B // _NW
_CHUNK = 1600
_NCHUNK = _BPW // _CHUNK


def _make_gather():
  mesh = plsc.VectorSubcoreMesh(core_axis_name="c", subcore_axis_name="s")

  @functools.partial(
      pl.kernel,
      mesh=mesh,
      out_type=jax.ShapeDtypeStruct((_B, _D), jnp.float32),
      scratch_types=[
          pltpu.VMEM((_CHUNK,), jnp.int32),
          pltpu.VMEM((_CHUNK, _D), jnp.float32),
          pltpu.SemaphoreType.DMA,
      ],
      compiler_params=pltpu.CompilerParams(use_tc_tiling_on_sc=False),
  )
  def gather_kernel(table_hbm, idx_hbm, out_hbm, idx_v, rows_v, sem):
    wid = lax.axis_index("s") * _NC + lax.axis_index("c")
    base = wid * _BPW

    def body(i, carry):
      off = base + i * _CHUNK
      pltpu.sync_copy(idx_hbm.at[pl.ds(off, _CHUNK)], idx_v)
      pltpu.async_copy(table_hbm.at[idx_v], rows_v, sem).wait()
      pltpu.sync_copy(rows_v, out_hbm.at[pl.ds(off, _CHUNK)])
      return carry

    lax.fori_loop(0, _NCHUNK, body, 0)

  return gather_kernel


_gather = _make_gather()


def kernel(x, table):
  idx = x.reshape(_B)
  out = _gather(table, idx)
  return out.reshape(_BATCH, _HIST, _D)


# speedup vs baseline: 1.1120x; 1.0097x over previous
import functools

import jax
import jax.numpy as jnp
from jax import lax
from jax.experimental import pallas as pl
from jax.experimental.pallas import tpu as pltpu
from jax.experimental.pallas import tpu_sc as plsc

_BATCH = 16384
_HIST = 50
_D = 32
_B = _BATCH * _HIST

_INFO = plsc.get_sparse_core_info()
_NC = _INFO.num_cores
_NS = _INFO.num_subcores
_NW = _NC * _NS
_BPW = _B // _NW
_CHUNK = 1600
_NCHUNK = _BPW // _CHUNK


def _make_gather():
  mesh = plsc.VectorSubcoreMesh(core_axis_name="c", subcore_axis_name="s")

  @functools.partial(
      pl.kernel,
      mesh=mesh,
      out_type=jax.ShapeDtypeStruct((_B, _D), jnp.float32),
      scratch_types=[
          pltpu.VMEM((2, _CHUNK), jnp.int32),
          pltpu.VMEM((2, _CHUNK, _D), jnp.float32),
          pltpu.SemaphoreType.DMA,
          pltpu.SemaphoreType.DMA,
          pltpu.SemaphoreType.DMA,
          pltpu.SemaphoreType.DMA,
          pltpu.SemaphoreType.DMA,
          pltpu.SemaphoreType.DMA,
      ],
      compiler_params=pltpu.CompilerParams(use_tc_tiling_on_sc=False),
  )
  def gather_kernel(table_hbm, idx_hbm, out_hbm, idx_v, rows_v,
                    si0, si1, sg0, sg1, so0, so1):
    si = (si0, si1)
    sg = (sg0, sg1)
    so = (so0, so1)
    wid = lax.axis_index("s") * _NC + lax.axis_index("c")
    base = wid * _BPW

    def idx_start(c, b):
      pltpu.async_copy(idx_hbm.at[pl.ds(base + c * _CHUNK, _CHUNK)],
                       idx_v.at[b], si[b])

    def idx_wait(c, b):
      pltpu.make_async_copy(idx_hbm.at[pl.ds(base + c * _CHUNK, _CHUNK)],
                            idx_v.at[b], si[b]).wait()

    def g_start(b):
      pltpu.async_copy(table_hbm.at[idx_v.at[b]], rows_v.at[b], sg[b])

    def g_wait(b):
      pltpu.make_async_copy(table_hbm.at[idx_v.at[b]], rows_v.at[b],
                            sg[b]).wait()

    def out_start(c, b):
      pltpu.async_copy(rows_v.at[b],
                       out_hbm.at[pl.ds(base + c * _CHUNK, _CHUNK)], so[b])

    def out_wait(c, b):
      pltpu.make_async_copy(rows_v.at[b],
                            out_hbm.at[pl.ds(base + c * _CHUNK, _CHUNK)],
                            so[b]).wait()

    idx_start(0, 0)
    idx_start(1, 1)
    idx_wait(0, 0)
    g_start(0)
    for i in range(1, _NCHUNK):
      s = i % 2
      p = 1 - s
      idx_wait(i, s)
      if i >= 2:
        out_wait(i - 2, s)
      g_start(s)
      g_wait(p)
      out_start(i - 1, p)
      if i + 1 < _NCHUNK:
        idx_start(i + 1, p)
    last = (_NCHUNK - 1) % 2
    g_wait(last)
    out_start(_NCHUNK - 1, last)
    out_wait(_NCHUNK - 2, 1 - last)
    out_wait(_NCHUNK - 1, last)

  return gather_kernel


_gather = _make_gather()


def kernel(x, table):
  idx = x.reshape(_B)
  out = _gather(table, idx)
  return out.reshape(_BATCH, _HIST, _D)
